# baseline (speedup 1.0000x reference)
import functools

import jax
import jax.numpy as jnp
from jax import lax
from jax.experimental import pallas as pl
from jax.experimental.pallas import tpu as pltpu
from jax.experimental.pallas import tpu_sc as plsc

_SEQ = 8192
_DIM = 1024
_LANES = 16
_NC = 2
_NS = 16
_NW = _NC * _NS
_ROWS_PER_W = _SEQ // _NW
_CHUNK = 32
_N_CHUNKS = _ROWS_PER_W // _CHUNK
_NBUF = 3
_VECS_PER_CHUNK = _CHUNK * _DIM // _LANES
_SCALE = _DIM ** -0.5


def _sc_embed(emb, idx):
    mesh = plsc.VectorSubcoreMesh(
        core_axis_name="c", subcore_axis_name="s",
        num_cores=_NC, num_subcores=_NS)

    @functools.partial(
        pl.kernel,
        out_type=jax.ShapeDtypeStruct((_SEQ, _DIM), jnp.float32),
        mesh=mesh,
        scratch_types=[
            pltpu.VMEM((_ROWS_PER_W,), jnp.int32),
            [pltpu.VMEM((_CHUNK, _DIM), jnp.float32)] * _NBUF,
            [pltpu.SemaphoreType.DMA] * _NBUF,
            [pltpu.SemaphoreType.DMA] * _NBUF,
        ],
    )
    def body(emb_hbm, idx_hbm, out_hbm, idx_v, bufs, gsems, ssems):
        wid = lax.axis_index("s") * _NC + lax.axis_index("c")
        base = wid * _ROWS_PER_W
        pltpu.sync_copy(idx_hbm.at[pl.ds(base, _ROWS_PER_W)], idx_v)

        def gather(ci):
            b = ci % _NBUF
            return pltpu.async_copy(
                emb_hbm.at[idx_v.at[pl.ds(ci * _CHUNK, _CHUNK)]],
                bufs[b], gsems[b])

        def scatter(ci):
            b = ci % _NBUF
            return pltpu.async_copy(
                bufs[b], out_hbm.at[pl.ds(base + ci * _CHUNK, _CHUNK)],
                ssems[b])

        gd = {}
        sd = {}
        for ci in range(_NBUF - 1):
            gd[ci] = gather(ci)
        for ci in range(_N_CHUNKS):
            nxt = ci + _NBUF - 1
            if nxt < _N_CHUNKS:
                if ci > 0:
                    sd[ci - 1].wait()
                gd[nxt] = gather(nxt)
            gd[ci].wait()
            buf = bufs[ci % _NBUF]

            @plsc.parallel_loop(0, _VECS_PER_CHUNK, unroll=8)
            def _scale_vec(k):
                r = k // (_DIM // _LANES)
                j = k % (_DIM // _LANES)
                sl = pl.ds(j * _LANES, _LANES)
                buf[r, sl] = buf[r, sl] * _SCALE

            sd[ci] = scatter(ci)
        for ci in range(_N_CHUNKS - _NBUF, _N_CHUNKS):
            sd[ci].wait()

    return body(emb, idx)


def kernel(x, pos, emb):
    del x
    return _sc_embed(emb, pos.astype(jnp.int32))


# speedup vs baseline: 1.0029x; 1.0029x over previous
import functools

import jax
import jax.numpy as jnp
from jax import lax
from jax.experimental import pallas as pl
from jax.experimental.pallas import tpu as pltpu
from jax.experimental.pallas import tpu_sc as plsc

_SEQ = 8192
_DIM = 1024
_LANES = 16
_NC = 2
_NS = 16
_NW = _NC * _NS
_ROWS_PER_W = _SEQ // _NW
_CHUNK = 32
_N_CHUNKS = _ROWS_PER_W // _CHUNK
_NBUF = 3
_VECS_PER_CHUNK = _CHUNK * _DIM // _LANES
_SCALE = _DIM ** -0.5


def _sc_embed(emb, idx):
    mesh = plsc.VectorSubcoreMesh(
        core_axis_name="c", subcore_axis_name="s",
        num_cores=_NC, num_subcores=_NS)

    @functools.partial(
        pl.kernel,
        out_type=jax.ShapeDtypeStruct((_SEQ, _DIM), jnp.float32),
        mesh=mesh,
        scratch_types=[
            pltpu.VMEM((_ROWS_PER_W,), jnp.int32),
            [pltpu.VMEM((_CHUNK, _DIM), jnp.float32)] * _NBUF,
            [pltpu.SemaphoreType.DMA] * _NBUF,
            [pltpu.SemaphoreType.DMA] * _NBUF,
        ],
    )
    def body(emb_hbm, idx_hbm, out_hbm, idx_v, bufs, gsems, ssems):
        wid = lax.axis_index("s") * _NC + lax.axis_index("c")
        base = wid * _ROWS_PER_W
        pltpu.sync_copy(idx_hbm.at[pl.ds(base, _ROWS_PER_W)], idx_v)

        def gather(ci):
            b = ci % _NBUF
            return pltpu.async_copy(
                emb_hbm.at[idx_v.at[pl.ds(ci * _CHUNK, _CHUNK)]],
                bufs[b], gsems[b])

        def scatter(ci):
            b = ci % _NBUF
            return pltpu.async_copy(
                bufs[b], out_hbm.at[pl.ds(base + ci * _CHUNK, _CHUNK)],
                ssems[b])

        gd = {}
        sd = {}
        for ci in range(_NBUF - 1):
            gd[ci] = gather(ci)
        for ci in range(_N_CHUNKS):
            nxt = ci + _NBUF - 1
            if nxt < _N_CHUNKS:
                if ci > 0:
                    sd[ci - 1].wait()
                gd[nxt] = gather(nxt)
            gd[ci].wait()
            buf = bufs[ci % _NBUF]

            @plsc.parallel_loop(0, _VECS_PER_CHUNK, unroll=16)
            def _scale_vec(k):
                r = k // (_DIM // _LANES)
                j = k % (_DIM // _LANES)
                sl = pl.ds(j * _LANES, _LANES)
                buf[r, sl] = buf[r, sl] * _SCALE

            sd[ci] = scatter(ci)
        for ci in range(_N_CHUNKS - _NBUF, _N_CHUNKS):
            sd[ci].wait()

    return body(emb, idx)


def kernel(x, pos, emb):
    del x
    return _sc_embed(emb, pos.astype(jnp.int32))


# speedup vs baseline: 1.0100x; 1.0071x over previous
import functools

import jax
import jax.numpy as jnp
from jax import lax
from jax.experimental import pallas as pl
from jax.experimental.pallas import tpu as pltpu
from jax.experimental.pallas import tpu_sc as plsc

_SEQ = 8192
_DIM = 1024
_LANES = 16
_NC = 2
_NS = 16
_NW = _NC * _NS
_ROWS_PER_W = _SEQ // _NW
_CHUNK = 16
_N_CHUNKS = _ROWS_PER_W // _CHUNK
_NBUF = 6
_DEPTH = 3
_VECS_PER_CHUNK = _CHUNK * _DIM // _LANES
_SCALE = _DIM ** -0.5


def _sc_embed(emb, idx):
    mesh = plsc.VectorSubcoreMesh(
        core_axis_name="c", subcore_axis_name="s",
        num_cores=_NC, num_subcores=_NS)

    @functools.partial(
        pl.kernel,
        out_type=jax.ShapeDtypeStruct((_SEQ, _DIM), jnp.float32),
        mesh=mesh,
        scratch_types=[
            pltpu.VMEM((_ROWS_PER_W,), jnp.int32),
            [pltpu.VMEM((_CHUNK, _DIM), jnp.float32)] * _NBUF,
            [pltpu.SemaphoreType.DMA] * _NBUF,
            [pltpu.SemaphoreType.DMA] * _NBUF,
        ],
    )
    def body(emb_hbm, idx_hbm, out_hbm, idx_v, bufs, gsems, ssems):
        wid = lax.axis_index("s") * _NC + lax.axis_index("c")
        base = wid * _ROWS_PER_W
        pltpu.sync_copy(idx_hbm.at[pl.ds(base, _ROWS_PER_W)], idx_v)

        def gather(ci):
            b = ci % _NBUF
            return pltpu.async_copy(
                emb_hbm.at[idx_v.at[pl.ds(ci * _CHUNK, _CHUNK)]],
                bufs[b], gsems[b])

        def scatter(ci):
            b = ci % _NBUF
            return pltpu.async_copy(
                bufs[b], out_hbm.at[pl.ds(base + ci * _CHUNK, _CHUNK)],
                ssems[b])

        gd = {}
        sd = {}
        for ci in range(_DEPTH):
            gd[ci] = gather(ci)
        for ci in range(_N_CHUNKS):
            nxt = ci + _DEPTH
            if nxt < _N_CHUNKS:
                prev = nxt - _NBUF
                if prev >= 0:
                    sd[prev].wait()
                gd[nxt] = gather(nxt)
            gd[ci].wait()
            buf = bufs[ci % _NBUF]

            @plsc.parallel_loop(0, _VECS_PER_CHUNK, unroll=16)
            def _scale_vec(k):
                r = k // (_DIM // _LANES)
                j = k % (_DIM // _LANES)
                sl = pl.ds(j * _LANES, _LANES)
                buf[r, sl] = buf[r, sl] * _SCALE

            sd[ci] = scatter(ci)
        for ci in range(_N_CHUNKS - _NBUF, _N_CHUNKS):
            sd[ci].wait()

    return body(emb, idx)


def kernel(x, pos, emb):
    del x
    return _sc_embed(emb, pos.astype(jnp.int32))


# speedup vs baseline: 1.0302x; 1.0200x over previous
import functools

import jax
import jax.numpy as jnp
from jax import lax
from jax.experimental import pallas as pl
from jax.experimental.pallas import tpu as pltpu
from jax.experimental.pallas import tpu_sc as plsc

_SEQ = 8192
_DIM = 1024
_LANES = 16
_NC = 2
_NS = 16
_NW = _NC * _NS
_ROWS_PER_W = _SEQ // _NW
_CHUNK = 16
_N_CHUNKS = _ROWS_PER_W // _CHUNK
_NBUF = 6
_DEPTH = 4
_VECS_PER_CHUNK = _CHUNK * _DIM // _LANES
_SCALE = _DIM ** -0.5


def _sc_embed(emb, idx):
    mesh = plsc.VectorSubcoreMesh(
        core_axis_name="c", subcore_axis_name="s",
        num_cores=_NC, num_subcores=_NS)

    @functools.partial(
        pl.kernel,
        out_type=jax.ShapeDtypeStruct((_SEQ, _DIM), jnp.float32),
        mesh=mesh,
        scratch_types=[
            pltpu.VMEM((_ROWS_PER_W,), jnp.int32),
            [pltpu.VMEM((_CHUNK, _DIM), jnp.float32)] * _NBUF,
            [pltpu.SemaphoreType.DMA] * _NBUF,
            [pltpu.SemaphoreType.DMA] * _NBUF,
            [pltpu.SemaphoreType.DMA] * 2,
        ],
    )
    def body(emb_hbm, idx_hbm, out_hbm, idx_v, bufs, gsems, ssems, isems):
        wid = lax.axis_index("s") * _NC + lax.axis_index("c")
        base = wid * _ROWS_PER_W
        head = _CHUNK * _DEPTH
        idx_head = pltpu.async_copy(
            idx_hbm.at[pl.ds(base, head)], idx_v.at[pl.ds(0, head)], isems[0])
        idx_tail = pltpu.async_copy(
            idx_hbm.at[pl.ds(base + head, _ROWS_PER_W - head)],
            idx_v.at[pl.ds(head, _ROWS_PER_W - head)], isems[1])

        def gather(ci):
            b = ci % _NBUF
            return pltpu.async_copy(
                emb_hbm.at[idx_v.at[pl.ds(ci * _CHUNK, _CHUNK)]],
                bufs[b], gsems[b])

        def scatter(ci):
            b = ci % _NBUF
            return pltpu.async_copy(
                bufs[b], out_hbm.at[pl.ds(base + ci * _CHUNK, _CHUNK)],
                ssems[b])

        gd = {}
        sd = {}
        idx_head.wait()
        for ci in range(_DEPTH):
            gd[ci] = gather(ci)
        idx_tail.wait()
        for ci in range(_N_CHUNKS):
            nxt = ci + _DEPTH
            if nxt < _N_CHUNKS:
                prev = nxt - _NBUF
                if prev >= 0:
                    sd[prev].wait()
                gd[nxt] = gather(nxt)
            gd[ci].wait()
            buf = bufs[ci % _NBUF]

            @plsc.parallel_loop(0, _VECS_PER_CHUNK, unroll=16)
            def _scale_vec(k):
                r = k // (_DIM // _LANES)
                j = k % (_DIM // _LANES)
                sl = pl.ds(j * _LANES, _LANES)
                buf[r, sl] = buf[r, sl] * _SCALE

            sd[ci] = scatter(ci)
        for ci in range(_N_CHUNKS - _NBUF, _N_CHUNKS):
            sd[ci].wait()

    return body(emb, idx)


def kernel(x, pos, emb):
    del x
    return _sc_embed(emb, pos.astype(jnp.int32))
